# baseline (speedup 1.0000x reference)
import jax
import jax.numpy as jnp
from jax.experimental import pallas as pl

_MAX_WEIGHT_NORM = 0.01


def _noise_kernel(special_ref, common_ref, out_ref):
    s = special_ref[...]
    sq = jnp.sum(s * s)
    norm = jnp.sqrt(sq)
    scale = jnp.where(norm > _MAX_WEIGHT_NORM, _MAX_WEIGHT_NORM / norm, 1.0)
    out_ref[...] = s * scale + common_ref[...]


def kernel(x, lens, common_latent, special_latent):
    del x, lens
    return pl.pallas_call(
        _noise_kernel,
        out_shape=jax.ShapeDtypeStruct(common_latent.shape, common_latent.dtype),
    )(special_latent, common_latent)
